# baseline (speedup 1.0000x reference)
import jax
import jax.numpy as jnp
from jax.experimental import pallas as pl


def _relu_body(x_ref, o_ref):
    o_ref[...] = jnp.maximum(x_ref[...], 0.0)


def kernel(inputs):
    b0, b1, h, w, c = inputs.shape
    xt = inputs.transpose(0, 1, 2, 4, 3)
    block_h = 28
    spec = pl.BlockSpec(
        (1, 1, block_h, c, w), lambda i, j, k: (i, j, k, 0, 0)
    )
    out = pl.pallas_call(
        _relu_body,
        grid=(b0, b1, h // block_h),
        in_specs=[spec],
        out_specs=spec,
        out_shape=jax.ShapeDtypeStruct(xt.shape, jnp.float32),
    )(xt)
    return out.transpose(0, 1, 2, 4, 3)
